# baseline (device time: 26118 ns/iter reference)
import functools

import jax
import jax.numpy as jnp
from jax import lax
from jax.experimental import pallas as pl
from jax.experimental.pallas import tpu as pltpu

N_DEV = 4
N_HOP = N_DEV - 1
S = 2


def kernel(x, w_mat):
    m_global, k_per = x.shape
    _, n = w_mat.shape
    m_per = m_global // N_DEV
    nh = n // 2
    ns = nh // S

    def body(x_hbm, w_hbm, out_ref, *scratch):
        comm = scratch[0:4]
        ssems = scratch[4:8]
        rsems = scratch[8:12]
        x_ref, w_ref = scratch[12], scratch[13]
        load_sems = scratch[14]

        p = lax.axis_index("i")
        left = lax.rem(p + N_DEV - 1, N_DEV)
        right = lax.rem(p + 1, N_DEV)
        targets = [right, right, left, left]

        barrier_sem = pltpu.get_barrier_semaphore()
        for nbr in [left, right]:
            pl.semaphore_signal(
                barrier_sem, inc=1,
                device_id=(nbr,), device_id_type=pl.DeviceIdType.MESH,
            )

        dma_w = pltpu.make_async_copy(w_hbm, w_ref, load_sems.at[0])
        dma_w.start()
        chunk_order = [
            lax.rem(p + N_DEV - 1, N_DEV),
            lax.rem(p + 1, N_DEV),
            lax.rem(p + 2, N_DEV),
            p,
        ]
        dma_x = []
        for i, c in enumerate(chunk_order):
            rows = pl.ds(c * m_per, m_per)
            d = pltpu.make_async_copy(
                x_hbm.at[rows, :], x_ref.at[rows, :], load_sems.at[1 + i]
            )
            d.start()
            dma_x.append(d)

        rdmas = {}
        for t in range(N_HOP):
            for k in range(4):
                rdmas[(t, k)] = pltpu.make_async_remote_copy(
                    src_ref=comm[k].at[t],
                    dst_ref=comm[k].at[t + 1],
                    send_sem=ssems[k].at[t],
                    recv_sem=rsems[k].at[t],
                    device_id=(targets[k],),
                    device_id_type=pl.DeviceIdType.MESH,
                )

        def partial_r(c):
            return jnp.dot(
                x_ref[pl.ds(c * m_per, m_per), :], w_ref[:, pl.ds(0, nh)],
                preferred_element_type=jnp.float32,
            )

        def partial_l(c):
            return jnp.dot(
                x_ref[pl.ds(c * m_per, m_per), :], w_ref[:, pl.ds(nh, nh)],
                preferred_element_type=jnp.float32,
            )

        dma_w.wait()
        dma_x[0].wait()
        seed_r = partial_r(chunk_order[0])
        comm[0][0, :, :] = seed_r[:, 0:ns]
        comm[1][0, :, :] = seed_r[:, ns:2 * ns]
        dma_x[1].wait()
        seed_l = partial_l(chunk_order[1])
        comm[2][0, :, :] = seed_l[:, 0:ns]
        comm[3][0, :, :] = seed_l[:, ns:2 * ns]

        pl.semaphore_wait(barrier_sem, 2)
        for k in range(4):
            rdmas[(0, k)].start()

        for t in range(N_HOP):
            if t == 0:
                dma_x[2].wait()
            elif t == N_HOP - 1:
                dma_x[3].wait()
            part_r = partial_r(lax.rem(p + 2 * N_DEV - 2 - t, N_DEV))
            part_l = partial_l(lax.rem(p + 2 + t, N_DEV))
            halves = [
                (0, part_r[:, 0:ns], 0),
                (2, part_l[:, 0:ns], nh),
                (1, part_r[:, ns:2 * ns], ns),
                (3, part_l[:, ns:2 * ns], nh + ns),
            ]
            for k, part, out_lo in halves:
                rdmas[(t, k)].wait_recv()
                acc = comm[k][t + 1, :, :] + part
                if t < N_HOP - 1:
                    comm[k][t + 1, :, :] = acc
                    rdmas[(t + 1, k)].start()
                else:
                    out_ref[:, pl.ds(out_lo, ns)] = acc

        for t in range(N_HOP):
            for k in range(4):
                rdmas[(t, k)].wait_send()

        @functools.partial(
            pl.run_scoped, second_barrier=pltpu.SemaphoreType.REGULAR
        )
        def _(second_barrier):
            for nbr in [left, right]:
                pl.semaphore_signal(
                    second_barrier, inc=1,
                    device_id=(nbr,), device_id_type=pl.DeviceIdType.MESH,
                )
            pl.semaphore_wait(second_barrier, 2)

    return pl.pallas_call(
        body,
        out_shape=jax.ShapeDtypeStruct((m_per, n), jnp.float32),
        in_specs=[
            pl.BlockSpec(memory_space=pl.ANY),
            pl.BlockSpec(memory_space=pl.ANY),
        ],
        out_specs=pl.BlockSpec(memory_space=pltpu.VMEM),
        scratch_shapes=(
            [pltpu.VMEM((N_HOP + 1, m_per, ns), jnp.float32)] * 4
            + [pltpu.SemaphoreType.DMA((N_HOP,))] * 8
            + [
                pltpu.VMEM((m_global, k_per), jnp.float32),
                pltpu.VMEM((k_per, n), jnp.float32),
                pltpu.SemaphoreType.DMA((5,)),
            ]
        ),
        compiler_params=pltpu.CompilerParams(collective_id=0),
    )(x, w_mat)


# device time: 25709 ns/iter; 1.0159x vs baseline; 1.0159x over previous
import jax
import jax.numpy as jnp
from jax import lax
from jax.experimental import pallas as pl
from jax.experimental.pallas import tpu as pltpu

N_DEV = 4
N_HOP = N_DEV - 1
S = 2


def kernel(x, w_mat):
    m_global, k_per = x.shape
    _, n = w_mat.shape
    m_per = m_global // N_DEV
    nh = n // 2
    ns = nh // S

    def body(x_hbm, w_hbm, out_ref, *scratch):
        comm = scratch[0:4]
        ssems = scratch[4:8]
        rsems = scratch[8:12]
        x_ref, w_ref = scratch[12], scratch[13]
        load_sems = scratch[14]

        p = lax.axis_index("i")
        left = lax.rem(p + N_DEV - 1, N_DEV)
        right = lax.rem(p + 1, N_DEV)
        targets = [right, right, left, left]

        barrier_sem = pltpu.get_barrier_semaphore()
        for nbr in [left, right]:
            pl.semaphore_signal(
                barrier_sem, inc=1,
                device_id=(nbr,), device_id_type=pl.DeviceIdType.MESH,
            )

        dma_w = pltpu.make_async_copy(w_hbm, w_ref, load_sems.at[0])
        dma_w.start()
        chunk_order = [
            lax.rem(p + N_DEV - 1, N_DEV),
            lax.rem(p + 1, N_DEV),
            lax.rem(p + 2, N_DEV),
            p,
        ]
        dma_x = []
        for i, c in enumerate(chunk_order):
            rows = pl.ds(c * m_per, m_per)
            d = pltpu.make_async_copy(
                x_hbm.at[rows, :], x_ref.at[rows, :], load_sems.at[1 + i]
            )
            d.start()
            dma_x.append(d)

        rdmas = {}
        for t in range(N_HOP):
            for k in range(4):
                rdmas[(t, k)] = pltpu.make_async_remote_copy(
                    src_ref=comm[k].at[t],
                    dst_ref=comm[k].at[t + 1],
                    send_sem=ssems[k].at[t],
                    recv_sem=rsems[k].at[t],
                    device_id=(targets[k],),
                    device_id_type=pl.DeviceIdType.MESH,
                )

        def partial_r(c):
            return jnp.dot(
                x_ref[pl.ds(c * m_per, m_per), :], w_ref[:, pl.ds(0, nh)],
                preferred_element_type=jnp.float32,
            )

        def partial_l(c):
            return jnp.dot(
                x_ref[pl.ds(c * m_per, m_per), :], w_ref[:, pl.ds(nh, nh)],
                preferred_element_type=jnp.float32,
            )

        dma_w.wait()
        dma_x[0].wait()
        dma_x[1].wait()
        barrier_waited = False
        seeds = [
            (0, chunk_order[0], 0),
            (2, chunk_order[1], nh),
            (1, chunk_order[0], ns),
            (3, chunk_order[1], nh + ns),
        ]
        for k, c, w_lo in seeds:
            comm[k][0, :, :] = jnp.dot(
                x_ref[pl.ds(c * m_per, m_per), :], w_ref[:, pl.ds(w_lo, ns)],
                preferred_element_type=jnp.float32,
            )
            if not barrier_waited:
                pl.semaphore_wait(barrier_sem, 2)
                barrier_waited = True
            rdmas[(0, k)].start()

        for t in range(N_HOP):
            if t == 0:
                dma_x[2].wait()
            elif t == N_HOP - 1:
                dma_x[3].wait()
            part_r = partial_r(lax.rem(p + 2 * N_DEV - 2 - t, N_DEV))
            part_l = partial_l(lax.rem(p + 2 + t, N_DEV))
            halves = [
                (0, part_r[:, 0:ns], 0),
                (2, part_l[:, 0:ns], nh),
                (1, part_r[:, ns:2 * ns], ns),
                (3, part_l[:, ns:2 * ns], nh + ns),
            ]
            for k, part, out_lo in halves:
                rdmas[(t, k)].wait_recv()
                acc = comm[k][t + 1, :, :] + part
                if t < N_HOP - 1:
                    comm[k][t + 1, :, :] = acc
                    rdmas[(t + 1, k)].start()
                else:
                    out_ref[:, pl.ds(out_lo, ns)] = acc

        for t in range(N_HOP):
            for k in range(4):
                rdmas[(t, k)].wait_send()

    return pl.pallas_call(
        body,
        out_shape=jax.ShapeDtypeStruct((m_per, n), jnp.float32),
        in_specs=[
            pl.BlockSpec(memory_space=pl.ANY),
            pl.BlockSpec(memory_space=pl.ANY),
        ],
        out_specs=pl.BlockSpec(memory_space=pltpu.VMEM),
        scratch_shapes=(
            [pltpu.VMEM((N_HOP + 1, m_per, ns), jnp.float32)] * 4
            + [pltpu.SemaphoreType.DMA((N_HOP,))] * 8
            + [
                pltpu.VMEM((m_global, k_per), jnp.float32),
                pltpu.VMEM((k_per, n), jnp.float32),
                pltpu.SemaphoreType.DMA((5,)),
            ]
        ),
        compiler_params=pltpu.CompilerParams(collective_id=0),
    )(x, w_mat)
